# baseline (speedup 1.0000x reference)
import functools

import jax
import jax.numpy as jnp
from jax import lax
from jax.experimental import pallas as pl
from jax.experimental.pallas import tpu as pltpu
from jax.experimental.pallas import tpu_sc as plsc

VOCAB = 100000
EMB = 128
NCLASS = 9
BATCH = 16384
WIN = 5
CPAD = 16
WPAD = 8
PCOLS = WPAD * CPAD
MM_BLOCK = 20000
NW = 32
SAMPLES_PER_W = BATCH // NW
ROWS_PER_W = SAMPLES_PER_W * WIN


def _mm_body(t_ref, w_ref, o_ref):
    o_ref[...] = jnp.dot(t_ref[...], w_ref[...],
                         preferred_element_type=jnp.float32)


def _score_table(table, w3):
    return pl.pallas_call(
        _mm_body,
        grid=(VOCAB // MM_BLOCK,),
        in_specs=[
            pl.BlockSpec((MM_BLOCK, EMB), lambda i: (i, 0)),
            pl.BlockSpec((EMB, PCOLS), lambda i: (0, 0)),
        ],
        out_specs=pl.BlockSpec((MM_BLOCK, PCOLS), lambda i: (i, 0)),
        out_shape=jax.ShapeDtypeStruct((VOCAB, PCOLS), jnp.float32),
    )(table, w3)


@functools.partial(
    pl.kernel,
    out_type=jax.ShapeDtypeStruct((BATCH, CPAD), jnp.float32),
    mesh=plsc.VectorSubcoreMesh(core_axis_name="c", subcore_axis_name="s"),
    scratch_types=[
        pltpu.VMEM((ROWS_PER_W // 16, 16), jnp.int32),
        pltpu.VMEM((ROWS_PER_W,), jnp.int32),
        pltpu.VMEM((ROWS_PER_W, CPAD), jnp.float32),
        pltpu.VMEM((SAMPLES_PER_W, CPAD), jnp.float32),
        pltpu.VMEM((CPAD,), jnp.float32),
        pltpu.SemaphoreType.DMA,
    ],
    compiler_params=pltpu.CompilerParams(use_tc_tiling_on_sc=False),
)
def _sc_gather_reduce(x_hbm, p_hbm, b_hbm, out_hbm,
                      x_v, idx_v, rows_v, out_v, bias_v, sem):
    wid = lax.axis_index("s") * 2 + lax.axis_index("c")
    base = wid * SAMPLES_PER_W
    pltpu.sync_copy(x_hbm.at[pl.ds(base * WIN // 16, ROWS_PER_W // 16)], x_v)
    pltpu.sync_copy(b_hbm, bias_v)

    wpat = [jnp.mod(jnp.arange(16, dtype=jnp.int32) + m, 5) for m in range(5)]
    zero = jnp.zeros((16,), jnp.int32)
    vmax = jnp.full((16,), VOCAB - 1, jnp.int32)

    def idx_body(i, carry):
        for m in range(5):
            k = i * 5 + m
            xv = jnp.minimum(jnp.maximum(x_v[k], zero), vmax)
            idx_v[pl.ds(k * 16, 16)] = xv * WPAD + wpat[m]
        return carry

    lax.fori_loop(0, ROWS_PER_W // (16 * 5), idx_body, 0)
    pltpu.async_copy(p_hbm.at[idx_v], rows_v, sem).wait()
    bias = bias_v[...]

    def body(i, carry):
        n = i * 4
        for u in range(4):
            k = (n + u) * WIN
            acc = bias + rows_v[k]
            acc = acc + rows_v[k + 1]
            acc = acc + rows_v[k + 2]
            acc = acc + rows_v[k + 3]
            acc = acc + rows_v[k + 4]
            out_v[n + u] = acc
        return carry

    lax.fori_loop(0, SAMPLES_PER_W // 4, body, 0)
    pltpu.sync_copy(out_v, out_hbm.at[pl.ds(base, SAMPLES_PER_W)])


def kernel(x, table, W, b):
    w3 = W.reshape(NCLASS, WIN, EMB).transpose(2, 1, 0)
    w3 = jnp.pad(w3, ((0, 0), (0, WPAD - WIN), (0, CPAD - NCLASS)))
    w3 = w3.reshape(EMB, PCOLS)
    b16 = jnp.pad(b, (0, CPAD - NCLASS))

    p = _score_table(table, w3)
    p = p.reshape(VOCAB * WPAD, CPAD)

    out = _sc_gather_reduce(x.reshape(BATCH * WIN // 16, 16), p, b16)
    return out[:, :NCLASS]
